# baseline (speedup 1.0000x reference)
import functools

import jax
import jax.numpy as jnp
from jax import lax
from jax.experimental import pallas as pl
from jax.experimental.pallas import tpu as pltpu
import jax.experimental.pallas.tpu_sc as plsc

N_NODES = 10000
D = 128
EPS = 0.1

NC = 2
NS = 16
NW = NC * NS

CHUNK = 128
N_PAD = 10112
N_CHUNKS_NODES = N_PAD // CHUNK


def _agg_body(n_chunks, x_hbm, rec_hbm, w_hbm, out_hbm,
              w_v, i0, i1, b0, b1, acc,
              g0, g1, ps0, ps1):
    bufs = (b0, b1)
    idxb = (i0, i1)
    gsems = (g0, g1)
    psems = (ps0, ps1)
    cid = lax.axis_index("c")
    sid = lax.axis_index("s")
    wid = sid * NC + cid

    for i in range(CHUNK):
        for j in range(D // 16):
            b0[i, pl.ds(j * 16, 16)] = jnp.zeros((16,), jnp.float32)
    for k in range((N_CHUNKS_NODES + NS - 1) // NS):
        node_chunk = sid + NS * k
        @pl.when(node_chunk < N_CHUNKS_NODES)
        def _():
            pltpu.sync_copy(b0, acc.at[pl.ds(node_chunk * CHUNK, CHUNK)])

    pltpu.sync_copy(w_hbm.at[wid], w_v)
    pltpu.async_copy(rec_hbm.at[wid, 0], i0, ps0)
    pltpu.async_copy(rec_hbm.at[wid, 1], i1, ps1)
    plsc.subcore_barrier()

    def _scale(rows, t):
        def _group(g, _c):
            wvec = w_v[t, pl.ds(g * 16, 16)]
            for e in range(16):
                row = g * 16 + e
                wv = jnp.full((16,), wvec[e], jnp.float32)
                for j in range(D // 16):
                    rows[row, pl.ds(j * 16, 16)] = (
                        rows[row, pl.ds(j * 16, 16)] * wv)
            return 0
        lax.fori_loop(0, CHUNK // 16, _group, 0)

    pltpu.make_async_copy(rec_hbm.at[wid, 0], i0, ps0).wait()
    pltpu.async_copy(x_hbm.at[i0.at[0]], b0, g0)

    def _pipe(p, _):
        for b in range(2):
            t = 2 * p + b
            nb = 1 - b
            pltpu.make_async_copy(rec_hbm.at[wid, t + 1], idxb[nb],
                                  psems[nb]).wait()
            pltpu.async_copy(x_hbm.at[idxb[nb].at[0]], bufs[nb], gsems[nb])
            pltpu.make_async_copy(x_hbm.at[idxb[b].at[0]], bufs[b],
                                  gsems[b]).wait()
            _scale(bufs[b], t)
            pltpu.sync_copy(bufs[b], acc.at[idxb[b].at[1]], add=True)
            pltpu.async_copy(rec_hbm.at[wid, t + 2], idxb[b], psems[b])
        return 0
    lax.fori_loop(0, n_chunks // 2, _pipe, 0)

    pltpu.make_async_copy(x_hbm.at[i0.at[0]], b0, g0).wait()
    pltpu.make_async_copy(rec_hbm.at[wid, 0], i1, ps1).wait()

    plsc.subcore_barrier()
    for k in range((N_CHUNKS_NODES + NS - 1) // NS):
        node_chunk = sid + NS * k
        @pl.when(node_chunk < N_CHUNKS_NODES)
        def _():
            pltpu.sync_copy(acc.at[pl.ds(node_chunk * CHUNK, CHUNK)],
                            out_hbm.at[cid, pl.ds(node_chunk * CHUNK, CHUNK)])


def _make_agg(n_chunks):
    mesh = plsc.VectorSubcoreMesh(core_axis_name="c", subcore_axis_name="s")
    return pl.kernel(
        functools.partial(_agg_body, n_chunks),
        out_type=jax.ShapeDtypeStruct((NC, N_PAD, D), jnp.float32),
        mesh=mesh,
        scratch_types=[
            pltpu.VMEM((n_chunks, CHUNK), jnp.float32),
            pltpu.VMEM((2, CHUNK), jnp.int32),
            pltpu.VMEM((2, CHUNK), jnp.int32),
            pltpu.VMEM((CHUNK, D), jnp.float32),
            pltpu.VMEM((CHUNK, D), jnp.float32),
            pltpu.VMEM_SHARED((N_PAD, D), jnp.float32),
            pltpu.SemaphoreType.DMA,
            pltpu.SemaphoreType.DMA,
            pltpu.SemaphoreType.DMA,
            pltpu.SemaphoreType.DMA,
        ],
    )


def _mlp_block(relu_out, x_ref, agg_ref, wa_ref, wb_ref, o_ref):
    h = (1.0 + EPS) * x_ref[...] + agg_ref[0] + agg_ref[1]
    h = jnp.maximum(jnp.dot(h, wa_ref[...], preferred_element_type=jnp.float32), 0.0)
    o = jnp.dot(h, wb_ref[...], preferred_element_type=jnp.float32)
    o_ref[...] = jnp.maximum(o, 0.0) if relu_out else o


def _mlp_call(x, agg, wa, wb, relu_out, blk=1000):
    nblk = N_NODES // blk
    return pl.pallas_call(
        functools.partial(_mlp_block, relu_out),
        grid=(nblk,),
        in_specs=[
            pl.BlockSpec((blk, D), lambda i: (i, 0)),
            pl.BlockSpec((NC, blk, D), lambda i: (0, i, 0)),
            pl.BlockSpec((D, D), lambda i: (0, 0)),
            pl.BlockSpec((D, D), lambda i: (0, 0)),
        ],
        out_specs=pl.BlockSpec((blk, D), lambda i: (i, 0)),
        out_shape=jax.ShapeDtypeStruct((N_NODES, D), jnp.float32),
    )(x, agg, wa, wb)


def kernel(x, edge_index, edge_weight, W1a, W1b, W2a, W2b):
    src = edge_index[0].astype(jnp.int32)
    dst = edge_index[1].astype(jnp.int32)
    w = edge_weight.astype(jnp.float32)

    n_edges = src.shape[0]
    per_tile = -(-n_edges // NW)
    n_chunks = 2 * (-(-per_tile // (2 * CHUNK)))
    e_pad = NW * n_chunks * CHUNK

    pad = e_pad - n_edges
    src_p = jnp.pad(src, (0, pad)).reshape(NW, n_chunks, CHUNK)
    dst_p = jnp.pad(dst, (0, pad)).reshape(NW, n_chunks, CHUNK)
    rec = jnp.stack([src_p, dst_p], axis=2)
    w_p = jnp.pad(w, (0, pad)).reshape(NW, n_chunks, CHUNK)
    rec = jnp.pad(rec, ((0, 0), (0, 2), (0, 0), (0, 0)))

    agg_fn = _make_agg(n_chunks)

    agg1 = agg_fn(x, rec, w_p)
    h = _mlp_call(x, agg1, W1a, W1b, relu_out=True)
    agg2 = agg_fn(h, rec, w_p)
    out = _mlp_call(h, agg2, W2a, W2b, relu_out=False)
    return out


# speedup vs baseline: 1.6766x; 1.6766x over previous
import functools

import jax
import jax.numpy as jnp
from jax import lax
from jax.experimental import pallas as pl
from jax.experimental.pallas import tpu as pltpu
import jax.experimental.pallas.tpu_sc as plsc

N_NODES = 10000
D = 128
EPS = 0.1

NC = 2
NS = 16
NW = NC * NS

CHUNK = 128
N_PAD = 10112
N_CHUNKS_NODES = N_PAD // CHUNK


def _agg_body(n_chunks, x_hbm, src_hbm, dst_hbm, w_hbm, out_hbm,
              src_v, dst_v, w_v, rows_v, acc, sem, sem2, sem3, sem4):
    cid = lax.axis_index("c")
    sid = lax.axis_index("s")
    wid = sid * NC + cid

    def _zrow(i, _):
        for j in range(D // 16):
            rows_v[i, pl.ds(j * 16, 16)] = jnp.zeros((16,), jnp.float32)
        return 0
    lax.fori_loop(0, CHUNK, _zrow, 0)
    for k in range((N_CHUNKS_NODES + NS - 1) // NS):
        node_chunk = sid + NS * k
        @pl.when(node_chunk < N_CHUNKS_NODES)
        def _():
            pltpu.sync_copy(rows_v, acc.at[pl.ds(node_chunk * CHUNK, CHUNK)])

    pltpu.sync_copy(src_hbm.at[wid], src_v)
    pltpu.sync_copy(dst_hbm.at[wid], dst_v)
    pltpu.sync_copy(w_hbm.at[wid], w_v)
    plsc.subcore_barrier()

    def _scale(rows, t):
        def _group(g, _c):
            wvec = w_v[t, pl.ds(g * 16, 16)]
            for e in range(16):
                row = g * 16 + e
                wv = jnp.full((16,), wvec[e], jnp.float32)
                for j in range(D // 16):
                    rows[row, pl.ds(j * 16, 16)] = (
                        rows[row, pl.ds(j * 16, 16)] * wv)
            return 0
        lax.fori_loop(0, CHUNK // 16, _group, 0)

    def _chunk(t, _):
        copies = []
        for q, s in ((0, sem), (1, sem2), (2, sem3), (3, sem4)):
            copies.append(pltpu.async_copy(
                x_hbm.at[src_v.at[t, pl.ds(q * 32, 32)]],
                rows_v.at[pl.ds(q * 32, 32)], s))
        for c in copies:
            c.wait()
        _scale(rows_v, t)
        pltpu.sync_copy(rows_v, acc.at[dst_v.at[t]], add=True)
        return 0
    lax.fori_loop(0, n_chunks, _chunk, 0)

    plsc.subcore_barrier()
    for k in range((N_CHUNKS_NODES + NS - 1) // NS):
        node_chunk = sid + NS * k
        @pl.when(node_chunk < N_CHUNKS_NODES)
        def _():
            pltpu.sync_copy(acc.at[pl.ds(node_chunk * CHUNK, CHUNK)],
                            out_hbm.at[cid, pl.ds(node_chunk * CHUNK, CHUNK)])


def _make_agg(n_chunks):
    mesh = plsc.VectorSubcoreMesh(core_axis_name="c", subcore_axis_name="s")
    return pl.kernel(
        functools.partial(_agg_body, n_chunks),
        out_type=jax.ShapeDtypeStruct((NC, N_PAD, D), jnp.float32),
        mesh=mesh,
        scratch_types=[
            pltpu.VMEM((n_chunks, CHUNK), jnp.int32),
            pltpu.VMEM((n_chunks, CHUNK), jnp.int32),
            pltpu.VMEM((n_chunks, CHUNK), jnp.float32),
            pltpu.VMEM((CHUNK, D), jnp.float32),
            pltpu.VMEM_SHARED((N_PAD, D), jnp.float32),
            pltpu.SemaphoreType.DMA,
            pltpu.SemaphoreType.DMA,
            pltpu.SemaphoreType.DMA,
            pltpu.SemaphoreType.DMA,
        ],
    )


def _mlp_block(relu_out, x_ref, agg_ref, wa_ref, wb_ref, o_ref):
    h = (1.0 + EPS) * x_ref[...] + agg_ref[0] + agg_ref[1]
    h = jnp.maximum(jnp.dot(h, wa_ref[...], preferred_element_type=jnp.float32), 0.0)
    o = jnp.dot(h, wb_ref[...], preferred_element_type=jnp.float32)
    o_ref[...] = jnp.maximum(o, 0.0) if relu_out else o


def _mlp_call(x, agg, wa, wb, relu_out, blk=1000):
    nblk = N_NODES // blk
    return pl.pallas_call(
        functools.partial(_mlp_block, relu_out),
        grid=(nblk,),
        in_specs=[
            pl.BlockSpec((blk, D), lambda i: (i, 0)),
            pl.BlockSpec((NC, blk, D), lambda i: (0, i, 0)),
            pl.BlockSpec((D, D), lambda i: (0, 0)),
            pl.BlockSpec((D, D), lambda i: (0, 0)),
        ],
        out_specs=pl.BlockSpec((blk, D), lambda i: (i, 0)),
        out_shape=jax.ShapeDtypeStruct((N_NODES, D), jnp.float32),
    )(x, agg, wa, wb)


def kernel(x, edge_index, edge_weight, W1a, W1b, W2a, W2b):
    src = edge_index[0].astype(jnp.int32)
    dst = edge_index[1].astype(jnp.int32)
    w = edge_weight.astype(jnp.float32)

    n_edges = src.shape[0]
    per_tile = -(-n_edges // NW)
    n_chunks = -(-per_tile // CHUNK)
    e_pad = NW * n_chunks * CHUNK

    pad = e_pad - n_edges
    src_p = jnp.pad(src, (0, pad)).reshape(NW, n_chunks, CHUNK)
    dst_p = jnp.pad(dst, (0, pad)).reshape(NW, n_chunks, CHUNK)
    w_p = jnp.pad(w, (0, pad)).reshape(NW, n_chunks, CHUNK)

    agg_fn = _make_agg(n_chunks)

    agg1 = agg_fn(x, src_p, dst_p, w_p)
    h = _mlp_call(x, agg1, W1a, W1b, relu_out=True)
    agg2 = agg_fn(h, src_p, dst_p, w_p)
    out = _mlp_call(h, agg2, W2a, W2b, relu_out=False)
    return out


# speedup vs baseline: 1.6937x; 1.0102x over previous
import functools

import jax
import jax.numpy as jnp
from jax import lax
from jax.experimental import pallas as pl
from jax.experimental.pallas import tpu as pltpu
import jax.experimental.pallas.tpu_sc as plsc

N_NODES = 10000
D = 128
EPS = 0.1

NC = 2
NS = 16
NW = NC * NS

CHUNK = 128
N_PAD = 10112
N_CHUNKS_NODES = N_PAD // CHUNK


def _agg_body(n_chunks, x_hbm, src_hbm, dst_hbm, w_hbm, out_hbm,
              src_v, dst_v, w_v, rows_v, acc, sem, sem2, sem3, sem4):
    cid = lax.axis_index("c")
    sid = lax.axis_index("s")
    wid = sid * NC + cid

    def _zrow(i, _):
        for j in range(D // 16):
            rows_v[i, pl.ds(j * 16, 16)] = jnp.zeros((16,), jnp.float32)
        return 0
    lax.fori_loop(0, CHUNK, _zrow, 0)
    for k in range((N_CHUNKS_NODES + NS - 1) // NS):
        node_chunk = sid + NS * k
        @pl.when(node_chunk < N_CHUNKS_NODES)
        def _():
            pltpu.sync_copy(rows_v, acc.at[pl.ds(node_chunk * CHUNK, CHUNK)])

    pltpu.sync_copy(src_hbm.at[wid], src_v)
    pltpu.sync_copy(dst_hbm.at[wid], dst_v)
    pltpu.sync_copy(w_hbm.at[wid], w_v)
    plsc.subcore_barrier()

    def _scale(rows, t, g_lo, g_hi):
        def _group(g, _c):
            wvec = w_v[t, pl.ds(g * 16, 16)]
            for e in range(16):
                row = g * 16 + e
                wv = jnp.full((16,), wvec[e], jnp.float32)
                for j in range(D // 16):
                    rows[row, pl.ds(j * 16, 16)] = (
                        rows[row, pl.ds(j * 16, 16)] * wv)
            return 0
        lax.fori_loop(g_lo, g_hi, _group, 0)

    def _chunk(t, _):
        copies = []
        for q, s in ((0, sem), (1, sem2), (2, sem3), (3, sem4)):
            copies.append(pltpu.async_copy(
                x_hbm.at[src_v.at[t, pl.ds(q * 32, 32)]],
                rows_v.at[pl.ds(q * 32, 32)], s))
        for q in range(4):
            copies[q].wait()
            _scale(rows_v, t, 2 * q, 2 * q + 2)
        pltpu.sync_copy(rows_v, acc.at[dst_v.at[t]], add=True)
        return 0
    lax.fori_loop(0, n_chunks, _chunk, 0)

    plsc.subcore_barrier()
    for k in range((N_CHUNKS_NODES + NS - 1) // NS):
        node_chunk = sid + NS * k
        @pl.when(node_chunk < N_CHUNKS_NODES)
        def _():
            pltpu.sync_copy(acc.at[pl.ds(node_chunk * CHUNK, CHUNK)],
                            out_hbm.at[cid, pl.ds(node_chunk * CHUNK, CHUNK)])


def _make_agg(n_chunks):
    mesh = plsc.VectorSubcoreMesh(core_axis_name="c", subcore_axis_name="s")
    return pl.kernel(
        functools.partial(_agg_body, n_chunks),
        out_type=jax.ShapeDtypeStruct((NC, N_PAD, D), jnp.float32),
        mesh=mesh,
        scratch_types=[
            pltpu.VMEM((n_chunks, CHUNK), jnp.int32),
            pltpu.VMEM((n_chunks, CHUNK), jnp.int32),
            pltpu.VMEM((n_chunks, CHUNK), jnp.float32),
            pltpu.VMEM((CHUNK, D), jnp.float32),
            pltpu.VMEM_SHARED((N_PAD, D), jnp.float32),
            pltpu.SemaphoreType.DMA,
            pltpu.SemaphoreType.DMA,
            pltpu.SemaphoreType.DMA,
            pltpu.SemaphoreType.DMA,
        ],
    )


def _mlp_block(relu_out, x_ref, agg_ref, wa_ref, wb_ref, o_ref):
    h = (1.0 + EPS) * x_ref[...] + agg_ref[0] + agg_ref[1]
    h = jnp.maximum(jnp.dot(h, wa_ref[...], preferred_element_type=jnp.float32), 0.0)
    o = jnp.dot(h, wb_ref[...], preferred_element_type=jnp.float32)
    o_ref[...] = jnp.maximum(o, 0.0) if relu_out else o


def _mlp_call(x, agg, wa, wb, relu_out, blk=1000):
    nblk = N_NODES // blk
    return pl.pallas_call(
        functools.partial(_mlp_block, relu_out),
        grid=(nblk,),
        in_specs=[
            pl.BlockSpec((blk, D), lambda i: (i, 0)),
            pl.BlockSpec((NC, blk, D), lambda i: (0, i, 0)),
            pl.BlockSpec((D, D), lambda i: (0, 0)),
            pl.BlockSpec((D, D), lambda i: (0, 0)),
        ],
        out_specs=pl.BlockSpec((blk, D), lambda i: (i, 0)),
        out_shape=jax.ShapeDtypeStruct((N_NODES, D), jnp.float32),
    )(x, agg, wa, wb)


def kernel(x, edge_index, edge_weight, W1a, W1b, W2a, W2b):
    src = edge_index[0].astype(jnp.int32)
    dst = edge_index[1].astype(jnp.int32)
    w = edge_weight.astype(jnp.float32)

    n_edges = src.shape[0]
    per_tile = -(-n_edges // NW)
    n_chunks = -(-per_tile // CHUNK)
    e_pad = NW * n_chunks * CHUNK

    pad = e_pad - n_edges
    src_p = jnp.pad(src, (0, pad)).reshape(NW, n_chunks, CHUNK)
    dst_p = jnp.pad(dst, (0, pad)).reshape(NW, n_chunks, CHUNK)
    w_p = jnp.pad(w, (0, pad)).reshape(NW, n_chunks, CHUNK)

    agg_fn = _make_agg(n_chunks)

    agg1 = agg_fn(x, src_p, dst_p, w_p)
    h = _mlp_call(x, agg1, W1a, W1b, relu_out=True)
    agg2 = agg_fn(h, src_p, dst_p, w_p)
    out = _mlp_call(h, agg2, W2a, W2b, relu_out=False)
    return out
